# baseline (speedup 1.0000x reference)
import functools

import jax
import jax.numpy as jnp
from jax import lax
from jax.experimental import pallas as pl
from jax.experimental.pallas import tpu as pltpu
from jax.experimental.pallas import tpu_sc as plsc

NC = 2
NS = 16
L = 16


def _prod_body(x_ref, out_ref):
    p = x_ref[...]
    sh = p.shape[1] // 2
    while sh >= 1:
        p = p * pltpu.roll(p, sh, axis=1)
        sh //= 2
    out_ref[...] = p[:, :1]


def _row_products(x, n, d, block_rows):
    return pl.pallas_call(
        _prod_body,
        grid=(n // block_rows,),
        in_specs=[pl.BlockSpec((block_rows, d), lambda i: (i, 0))],
        out_specs=pl.BlockSpec((block_rows, 1), lambda i: (i, 0)),
        out_shape=jax.ShapeDtypeStruct((n, 1), jnp.float32),
    )(x)


def _sc_segment_max(xp, edge_index, n, n_pad, e):
    nw = NC * NS
    c = ((e + nw * 128 - 1) // (nw * 128)) * 128
    groups = c // L
    U = 8
    assert groups % U == 0
    blocks_n = groups // U
    n_per_s = n_pad // NS

    mesh = plsc.VectorSubcoreMesh(
        core_axis_name="c", subcore_axis_name="s", num_cores=NC, num_subcores=NS
    )

    @functools.partial(
        pl.kernel,
        mesh=mesh,
        compiler_params=pltpu.CompilerParams(needs_layout_passes=False),
        out_type=jax.ShapeDtypeStruct((NC, n_pad), jnp.float32),
        scratch_types=[
            pltpu.VMEM((n_pad,), jnp.float32),
            pltpu.VMEM((n_pad,), jnp.float32),
            pltpu.VMEM((2, c), jnp.int32),
            pltpu.VMEM((NS, n_per_s), jnp.float32),
            pltpu.VMEM((n_per_s,), jnp.float32),
            pltpu.VMEM_SHARED((NS, n_pad), jnp.float32),
            pltpu.SMEM((blocks_n + 1,), jnp.int32),
            pltpu.SemaphoreType.DMA,
        ],
    )
    def k(xp_hbm, edge_hbm, out_hbm, xp_v, agg_v, sd_v, red_v, res_v, shared,
          wl_s, sem):
        cid = lax.axis_index("c")
        sid = lax.axis_index("s")
        wid = sid * NC + cid
        base = jnp.minimum(wid * c, e - c)

        cp1 = pltpu.async_copy(xp_hbm, xp_v.at[pl.ds(0, n)], sem)
        cp2 = pltpu.async_copy(xp_hbm, agg_v.at[pl.ds(0, n)], sem)
        cp3 = pltpu.async_copy(edge_hbm.at[:, pl.ds(base, c)], sd_v, sem)
        cp1.wait()
        cp2.wait()
        cp3.wait()

        def block(b, cnt):
            off0 = b * (U * L)
            dsts, vals = [], []
            for u in range(U):
                s16 = sd_v[0, pl.ds(off0 + u * L, L)]
                d16 = sd_v[1, pl.ds(off0 + u * L, L)]
                dsts.append(d16)
                vals.append(plsc.load_gather(xp_v, [s16]))
            for u in range(U):
                cur = plsc.load_gather(agg_v, [dsts[u]])
                plsc.store_scatter(agg_v, [dsts[u]],
                                   jnp.maximum(cur, vals[u]),
                                   mask=vals[u] > cur)
            lost = None
            for u in range(U):
                cur2 = plsc.load_gather(agg_v, [dsts[u]])
                l = vals[u] > cur2
                lost = l if lost is None else jnp.logical_or(lost, l)
            any_lost = jnp.any(lost)

            @pl.when(any_lost)
            def _():
                wl_s[cnt] = b

            return cnt + any_lost.astype(jnp.int32)

        cnt = lax.fori_loop(0, blocks_n, block, jnp.int32(0))

        def fix(kk, carry):
            b = wl_s[kk]
            off0 = b * (U * L)
            for u in range(U):
                s16 = sd_v[0, pl.ds(off0 + u * L, L)]
                d16 = sd_v[1, pl.ds(off0 + u * L, L)]
                val = plsc.load_gather(xp_v, [s16])

                def body(_, d16=d16, val=val):
                    cur = plsc.load_gather(agg_v, [d16])
                    plsc.store_scatter(agg_v, [d16], jnp.maximum(cur, val),
                                       mask=val > cur)
                    cur2 = plsc.load_gather(agg_v, [d16])
                    return jnp.any(val > cur2)

                lax.while_loop(lambda p: p, body, jnp.bool_(True))
            return carry

        lax.fori_loop(0, cnt, fix, 0)

        pltpu.sync_copy(agg_v, shared.at[sid])
        plsc.subcore_barrier()
        pltpu.sync_copy(shared.at[:, pl.ds(sid * n_per_s, n_per_s)], red_v)

        def red(v, carry):
            m = red_v[0, pl.ds(v * L, L)]
            for j in range(1, NS):
                m = jnp.maximum(m, red_v[j, pl.ds(v * L, L)])
            res_v[pl.ds(v * L, L)] = m
            return carry

        lax.fori_loop(0, n_per_s // L, red, 0)
        pltpu.sync_copy(res_v, out_hbm.at[cid, pl.ds(sid * n_per_s, n_per_s)])

    return k(xp, edge_index)


def _combine_body(p_ref, xp_ref, w_ref, out_ref):
    n = xp_ref.shape[1]
    agg = jnp.max(p_ref[...], axis=0)[:n]
    z = xp_ref[0, :] * w_ref[0, 0] + agg * w_ref[0, 1]
    out_ref[...] = z[None, :]


def _combine(partial, xp_row, weights, n):
    n_pad = partial.shape[1]
    return pl.pallas_call(
        _combine_body,
        in_specs=[
            pl.BlockSpec((NC, n_pad), lambda: (0, 0)),
            pl.BlockSpec((1, n), lambda: (0, 0)),
            pl.BlockSpec(memory_space=pltpu.SMEM),
        ],
        out_specs=pl.BlockSpec((1, n), lambda: (0, 0)),
        out_shape=jax.ShapeDtypeStruct((1, n), jnp.float32),
    )(partial, xp_row, weights)


def kernel(x, edge_index, weights):
    n, d = x.shape
    e = edge_index.shape[1]
    n_pad = 10240

    xp2d = _row_products(x, n, d, block_rows=2000)
    xp_flat = xp2d.reshape(n)
    partial = _sc_segment_max(xp_flat, edge_index, n, n_pad, e)
    z = _combine(partial, xp2d.reshape(1, n), weights, n)
    return z.reshape(n, 1)
